# baseline (speedup 1.0000x reference)
import functools

import jax
import jax.numpy as jnp
from jax import lax
from jax.experimental import pallas as pl
from jax.experimental.pallas import tpu as pltpu
from jax.experimental.pallas import tpu_sc as plsc

NC = 2
NS = 16
NW = NC * NS
LANES = 16
GRP = 128
CPAD = 128


def _make_sc_embed(N, L, TYPE_V, TOK_V, D):
    nodes_per_w = N // NW
    sc_nodes = 16
    n_steps = nodes_per_w // sc_nodes
    idx_per_step = sc_nodes * L
    n_gather = 4
    gather_idx = idx_per_step // n_gather
    idx_per_w = nodes_per_w * L

    mesh = plsc.VectorSubcoreMesh(
        core_axis_name="c", subcore_axis_name="s",
        num_cores=NC, num_subcores=NS)

    @functools.partial(
        pl.kernel,
        mesh=mesh,
        compiler_params=pltpu.CompilerParams(use_tc_tiling_on_sc=False),
        out_type=jax.ShapeDtypeStruct((N, CPAD), jnp.float32),
        scratch_types=(
            pltpu.VMEM((idx_per_w,), jnp.int32),
            pltpu.VMEM((nodes_per_w,), jnp.int32),
            pltpu.VMEM((idx_per_step, D), jnp.float32),
            pltpu.VMEM((idx_per_step, D), jnp.float32),
            pltpu.VMEM((sc_nodes, CPAD), jnp.float32),
            pltpu.VMEM((sc_nodes, CPAD), jnp.float32),
            pltpu.SemaphoreType.DMA,
            pltpu.SemaphoreType.DMA,
            pltpu.SemaphoreType.DMA,
            pltpu.SemaphoreType.DMA,
        ),
    )
    def sc_embed(tok_ids_hbm, type_ids_hbm, type_tab_hbm, tok_tab_hbm,
                 out_hbm,
                 tok_idx_v, type_idx_v, rows0, rows1, stage0, stage1,
                 sem0, sem1, semy0, semy1):
        wid = lax.axis_index("s") * NC + lax.axis_index("c")
        node_base = wid * nodes_per_w

        pltpu.sync_copy(tok_ids_hbm.at[pl.ds(wid * idx_per_w, idx_per_w)],
                        tok_idx_v)
        pltpu.sync_copy(type_ids_hbm.at[pl.ds(node_base, nodes_per_w)],
                        type_idx_v)

        def fire(s, rows, stage, sem, semy):
            pltpu.async_copy(
                type_tab_hbm.at[type_idx_v.at[pl.ds(s * sc_nodes, sc_nodes)]],
                stage, semy)
            for g in range(n_gather):
                pltpu.async_copy(
                    tok_tab_hbm.at[tok_idx_v.at[
                        pl.ds(s * idx_per_step + g * gather_idx, gather_idx)]],
                    rows.at[pl.ds(g * gather_idx, gather_idx)], sem)

        def drain(rows, stage, sem, semy):
            pltpu.make_async_copy(
                tok_tab_hbm.at[pl.ds(0, idx_per_step)], rows, sem).wait()
            pltpu.make_async_copy(
                type_tab_hbm.at[pl.ds(0, sc_nodes)], stage, semy).wait()

        def reduce_store(s, rows, stage):
            def node_body(i, ncarry):
                base = i * L
                for dv in range(D // LANES):
                    sl = pl.ds(dv * LANES, LANES)
                    acc = rows[base, sl]
                    for l in range(1, L):
                        acc = acc + rows[base + l, sl]
                    stage[i, pl.ds(D + dv * LANES, LANES)] = acc
                return ncarry
            lax.fori_loop(0, sc_nodes, node_body, 0)
            pltpu.sync_copy(
                stage, out_hbm.at[pl.ds(node_base + s * sc_nodes, sc_nodes)])

        fire(0, rows0, stage0, sem0, semy0)

        def pair_body(p, carry):
            s0 = 2 * p
            fire(s0 + 1, rows1, stage1, sem1, semy1)
            drain(rows0, stage0, sem0, semy0)
            reduce_store(s0, rows0, stage0)

            @pl.when(p < n_steps // 2 - 1)
            def _():
                fire(s0 + 2, rows0, stage0, sem0, semy0)
            drain(rows1, stage1, sem1, semy1)
            reduce_store(s0 + 1, rows1, stage1)
            return carry
        lax.fori_loop(0, n_steps // 2, pair_body, 0)

    return sc_embed


def _repack_body(ta_ref, tb_ref, i_ref, o_ref):
    dn = (((0,), (0,)), ((), ()))
    r1 = jax.lax.dot_general(ta_ref[...], i_ref[...], dn,
                             preferred_element_type=jnp.float32)
    r2 = jax.lax.dot_general(tb_ref[...], i_ref[...], dn,
                             preferred_element_type=jnp.float32)
    o_ref[:, : r1.shape[1]] = r1
    o_ref[:, r1.shape[1]:] = r2


HB = 16384


def _repack(table_t):
    D, V = table_t.shape
    eye = jnp.eye(D, dtype=jnp.float32)
    grid = (pl.cdiv(V, 2 * HB),)
    return pl.pallas_call(
        _repack_body,
        grid=grid,
        in_specs=[
            pl.BlockSpec((D, HB), lambda i: (0, 2 * i)),
            pl.BlockSpec((D, HB), lambda i: (0, 2 * i + 1)),
            pl.BlockSpec((D, D), lambda i: (0, 0)),
        ],
        out_specs=pl.BlockSpec((HB, 2 * D), lambda i: (i, 0)),
        out_shape=jax.ShapeDtypeStruct((grid[0] * HB, 2 * D), jnp.float32),
    )(table_t, table_t, eye)


def _mlp_body(x_ref, w_ref, b_ref, o_ref):
    y = jnp.dot(x_ref[...], w_ref[...],
                preferred_element_type=jnp.float32) + b_ref[...]
    o_ref[...] = jnp.maximum(y, 0.0)


def _mlp(x, wt, b2d):
    N, C = x.shape
    blk = 2048
    return pl.pallas_call(
        _mlp_body,
        grid=(N // blk,),
        in_specs=[
            pl.BlockSpec((blk, C), lambda i: (i, 0)),
            pl.BlockSpec((C, C), lambda i: (0, 0)),
            pl.BlockSpec((1, C), lambda i: (0, 0)),
        ],
        out_specs=pl.BlockSpec((blk, C), lambda i: (i, 0)),
        out_shape=jax.ShapeDtypeStruct((N, C), jnp.float32),
    )(x, wt, b2d)


def kernel(node_type_index, node_sub_token_ids, type_table, token_table, W, b):
    N, L = node_sub_token_ids.shape
    TYPE_V, D = type_table.shape
    TOK_V = token_table.shape[0]
    C = W.shape[0]

    ids = node_sub_token_ids.astype(jnp.int32).reshape(N * L)
    j = ids % (2 * HB)
    tok_ids = 2 * ((ids // (2 * HB)) * HB + j % HB) + j // HB
    type_ids = node_type_index.astype(jnp.int32)
    tok_lin = _repack(token_table.T).reshape(-1, D)
    type_pad = jnp.pad(type_table, ((0, 0), (0, CPAD - D)))

    sc_embed = _make_sc_embed(N, L, TYPE_V, TOK_V, D)
    node_emb = sc_embed(tok_ids, type_ids, type_pad, tok_lin)

    out = _mlp(node_emb, W.T, b.reshape(1, C))

    ast_node_index = jnp.arange(N, dtype=jnp.int32)
    return (ast_node_index, out)


# speedup vs baseline: 1.0023x; 1.0023x over previous
import functools

import jax
import jax.numpy as jnp
from jax import lax
from jax.experimental import pallas as pl
from jax.experimental.pallas import tpu as pltpu
from jax.experimental.pallas import tpu_sc as plsc

NC = 2
NS = 16
NW = NC * NS
LANES = 16
CPAD = 128


def _make_sc_embed(N, L, TYPE_V, TOK_V, D):
    nodes_per_w = N // NW
    sc_nodes = 16
    n_steps = nodes_per_w // sc_nodes
    idx_per_step = sc_nodes * L
    n_gather = 4
    gather_idx = idx_per_step // n_gather
    idx_per_w = nodes_per_w * L

    mesh = plsc.VectorSubcoreMesh(
        core_axis_name="c", subcore_axis_name="s",
        num_cores=NC, num_subcores=NS)

    @functools.partial(
        pl.kernel,
        mesh=mesh,
        compiler_params=pltpu.CompilerParams(use_tc_tiling_on_sc=False),
        out_type=jax.ShapeDtypeStruct((N, CPAD), jnp.float32),
        scratch_types=(
            pltpu.VMEM((idx_per_w,), jnp.int32),
            pltpu.VMEM((nodes_per_w,), jnp.int32),
            pltpu.VMEM((idx_per_step, D), jnp.float32),
            pltpu.VMEM((idx_per_step, D), jnp.float32),
            pltpu.VMEM((sc_nodes, CPAD), jnp.float32),
            pltpu.VMEM((sc_nodes, CPAD), jnp.float32),
            pltpu.SemaphoreType.DMA,
            pltpu.SemaphoreType.DMA,
            pltpu.SemaphoreType.DMA,
            pltpu.SemaphoreType.DMA,
        ),
    )
    def sc_embed(tok_ids_hbm, type_ids_hbm, type_tab_hbm, tok_tab_hbm,
                 out_hbm,
                 tok_idx_v, type_idx_v, rows0, rows1, stage0, stage1,
                 sem0, sem1, semy0, semy1):
        wid = lax.axis_index("s") * NC + lax.axis_index("c")
        node_base = wid * nodes_per_w

        pltpu.sync_copy(tok_ids_hbm.at[pl.ds(wid * idx_per_w, idx_per_w)],
                        tok_idx_v)
        pltpu.sync_copy(type_ids_hbm.at[pl.ds(node_base, nodes_per_w)],
                        type_idx_v)

        def fire(s, rows, stage, sem, semy):
            pltpu.async_copy(
                type_tab_hbm.at[type_idx_v.at[pl.ds(s * sc_nodes, sc_nodes)]],
                stage, semy)
            for g in range(n_gather):
                pltpu.async_copy(
                    tok_tab_hbm.at[tok_idx_v.at[
                        pl.ds(s * idx_per_step + g * gather_idx, gather_idx)]],
                    rows.at[pl.ds(g * gather_idx, gather_idx)], sem)

        def drain(rows, stage, sem, semy):
            pltpu.make_async_copy(
                tok_tab_hbm.at[pl.ds(0, idx_per_step)], rows, sem).wait()
            pltpu.make_async_copy(
                type_tab_hbm.at[pl.ds(0, sc_nodes)], stage, semy).wait()

        def reduce_store(s, rows, stage):
            def node_body(i, ncarry):
                base = i * L
                for dv in range(D // LANES):
                    sl = pl.ds(dv * LANES, LANES)
                    acc = rows[base, sl]
                    for l in range(1, L):
                        acc = acc + rows[base + l, sl]
                    stage[i, pl.ds(D + dv * LANES, LANES)] = acc
                return ncarry
            lax.fori_loop(0, sc_nodes, node_body, 0)
            pltpu.sync_copy(
                stage, out_hbm.at[pl.ds(node_base + s * sc_nodes, sc_nodes)])

        fire(0, rows0, stage0, sem0, semy0)

        def pair_body(p, carry):
            s0 = 2 * p
            fire(s0 + 1, rows1, stage1, sem1, semy1)
            drain(rows0, stage0, sem0, semy0)
            reduce_store(s0, rows0, stage0)

            @pl.when(p < n_steps // 2 - 1)
            def _():
                fire(s0 + 2, rows0, stage0, sem0, semy0)
            drain(rows1, stage1, sem1, semy1)
            reduce_store(s0 + 1, rows1, stage1)
            return carry
        lax.fori_loop(0, n_steps // 2, pair_body, 0)

    return sc_embed


def _repack_body(ta_ref, tb_ref, i_ref, o_ref):
    dn = (((0,), (0,)), ((), ()))
    r1 = jax.lax.dot_general(ta_ref[...], i_ref[...], dn,
                             preferred_element_type=jnp.float32)
    r2 = jax.lax.dot_general(tb_ref[...], i_ref[...], dn,
                             preferred_element_type=jnp.float32)
    o_ref[:, : r1.shape[1]] = r1
    o_ref[:, r1.shape[1]:] = r2


HB = 16384


def _repack(table_t):
    D, V = table_t.shape
    eye = jnp.eye(D, dtype=jnp.float32)
    grid = (pl.cdiv(V, 2 * HB),)
    return pl.pallas_call(
        _repack_body,
        grid=grid,
        in_specs=[
            pl.BlockSpec((D, HB), lambda i: (0, 2 * i)),
            pl.BlockSpec((D, HB), lambda i: (0, 2 * i + 1)),
            pl.BlockSpec((D, D), lambda i: (0, 0)),
        ],
        out_specs=pl.BlockSpec((HB, 2 * D), lambda i: (i, 0)),
        out_shape=jax.ShapeDtypeStruct((grid[0] * HB, 2 * D), jnp.float32),
    )(table_t, table_t, eye)


def _mlp_body(x_ref, w_ref, b_ref, o_ref):
    y = jnp.dot(x_ref[...], w_ref[...],
                preferred_element_type=jnp.float32) + b_ref[...]
    o_ref[...] = jnp.maximum(y, 0.0)


def _mlp(x, wt, b2d):
    N, C = x.shape
    blk = 2048
    return pl.pallas_call(
        _mlp_body,
        grid=(N // blk,),
        in_specs=[
            pl.BlockSpec((blk, C), lambda i: (i, 0)),
            pl.BlockSpec((C, C), lambda i: (0, 0)),
            pl.BlockSpec((1, C), lambda i: (0, 0)),
        ],
        out_specs=pl.BlockSpec((blk, C), lambda i: (i, 0)),
        out_shape=jax.ShapeDtypeStruct((N, C), jnp.float32),
    )(x, wt, b2d)


def kernel(node_type_index, node_sub_token_ids, type_table, token_table, W, b):
    N, L = node_sub_token_ids.shape
    TYPE_V, D = type_table.shape
    TOK_V = token_table.shape[0]
    C = W.shape[0]

    ids = node_sub_token_ids.astype(jnp.int32).reshape(N * L)
    j = ids % (2 * HB)
    tok_ids = 2 * ((ids // (2 * HB)) * HB + j % HB) + j // HB
    type_ids = node_type_index.astype(jnp.int32)
    tok_lin = _repack(token_table.T).reshape(-1, D)
    type_pad = jnp.pad(type_table, ((0, 0), (0, CPAD - D)))

    sc_embed = _make_sc_embed(N, L, TYPE_V, TOK_V, D)
    node_emb = sc_embed(tok_ids, type_ids, type_pad, tok_lin)

    out = _mlp(node_emb, W.T, b.reshape(1, C))

    ast_node_index = jnp.arange(N, dtype=jnp.int32)
    return (ast_node_index, out)
